# baseline (speedup 1.0000x reference)
import functools

import jax
import jax.numpy as jnp
from jax import lax
from jax.experimental import pallas as pl
from jax.experimental.pallas import tpu as pltpu
from jax.experimental.pallas import tpu_sc as plsc

_B = 16384
_DIM = 128
_NC = 2
_NS = 16
_NW = _NC * _NS
_HPW = _B // _NW
_RPW = 2 * _HPW
_CH = 128
_NCHUNK = _RPW // _CH
_L = 16
_G = _DIM // _L


def _lane_total(x):
    dnums = lax.GatherDimensionNumbers(
        offset_dims=(), collapsed_slice_dims=(0,), start_index_map=(0,))
    for sh in (8, 4, 2, 1):
        perm = lax.iota(jnp.int32, _L) ^ sh
        x = x + lax.gather(x, perm[:, None], dnums, slice_sizes=(1,),
                           mode=lax.GatherScatterMode.PROMISE_IN_BOUNDS)
    return x


def _sqrt16(x):
    i = plsc.bitcast(x, jnp.int32)
    g = plsc.bitcast(lax.shift_right_logical(i, 1) + 0x1FBD1DF5, jnp.float32)
    g = 0.5 * (g + x / g)
    g = 0.5 * (g + x / g)
    return g


@functools.partial(
    pl.kernel,
    mesh=plsc.VectorSubcoreMesh(core_axis_name="c", subcore_axis_name="s"),
    out_type=[jax.ShapeDtypeStruct((_B,), jnp.float32),
              jax.ShapeDtypeStruct((_B,), jnp.float32)],
    compiler_params=pltpu.CompilerParams(needs_layout_passes=False),
    scratch_types=[
        pltpu.VMEM((_RPW,), jnp.int32),
        pltpu.VMEM((_RPW,), jnp.int32),
        pltpu.VMEM((_RPW,), jnp.int32),
        pltpu.VMEM((2, _CH, _DIM), jnp.float32),
        pltpu.VMEM((2, _CH, _DIM), jnp.float32),
        pltpu.VMEM((2, _CH, _DIM), jnp.float32),
        pltpu.VMEM((_RPW,), jnp.float32),
        pltpu.SemaphoreType.DMA,
        pltpu.SemaphoreType.DMA,
        pltpu.SemaphoreType.DMA,
    ],
)
def _ttranse_sc(ps_hbm, pr_hbm, po_hbm, ns_hbm, nr_hbm, no_hbm,
                e_hbm, rel_hbm, pos_hbm, neg_hbm,
                sidx, ridx, oidx, sbuf, rbuf, obuf, res_v,
                sem0, sem1, semw):
    wid = lax.axis_index("s") * _NC + lax.axis_index("c")
    base = wid * _HPW
    sems = (sem0, sem1)
    ihp = [
        pltpu.async_copy(ps_hbm.at[pl.ds(base, _HPW)], sidx.at[pl.ds(0, _HPW)], sem0),
        pltpu.async_copy(pr_hbm.at[pl.ds(base, _HPW)], ridx.at[pl.ds(0, _HPW)], sem0),
        pltpu.async_copy(po_hbm.at[pl.ds(base, _HPW)], oidx.at[pl.ds(0, _HPW)], sem0),
    ]
    ihn = [
        pltpu.async_copy(ns_hbm.at[pl.ds(base, _HPW)], sidx.at[pl.ds(_HPW, _HPW)], semw),
        pltpu.async_copy(nr_hbm.at[pl.ds(base, _HPW)], ridx.at[pl.ds(_HPW, _HPW)], semw),
        pltpu.async_copy(no_hbm.at[pl.ds(base, _HPW)], oidx.at[pl.ds(_HPW, _HPW)], semw),
    ]
    for h in ihp:
        h.wait()
    lanes = lax.iota(jnp.int32, _L)

    def start(c):
        p = c & 1
        sem = sems[p]
        return (
            pltpu.async_copy(e_hbm.at[sidx.at[pl.ds(c * _CH, _CH)]],
                             sbuf.at[p], sem),
            pltpu.async_copy(rel_hbm.at[ridx.at[pl.ds(c * _CH, _CH)]],
                             rbuf.at[p], sem),
            pltpu.async_copy(e_hbm.at[oidx.at[pl.ds(c * _CH, _CH)]],
                             obuf.at[p], sem),
        )

    def half_done(h):
        @plsc.parallel_loop(0, _HPW // _L, unroll=4)
        def sqrt_body(i, h=h):
            off = h * _HPW + i * _L
            res_v[pl.ds(off, _L)] = _sqrt16(res_v[pl.ds(off, _L)])
        dst = pos_hbm if h == 0 else neg_hbm
        return pltpu.async_copy(res_v.at[pl.ds(h * _HPW, _HPW)],
                                dst.at[pl.ds(base, _HPW)], semw)

    wb = []
    pending = {0: start(0)}
    for c in range(_NCHUNK):
        p = c & 1
        if c + 1 < _NCHUNK:
            if c + 1 == _NCHUNK // 2:
                for h in ihn:
                    h.wait()
            pending[c + 1] = start(c + 1)
        for h in pending.pop(c):
            h.wait()

        @plsc.parallel_loop(0, _CH, unroll=6)
        def row_body(row, c=c, p=p):
            acc0 = jnp.zeros((_L,), jnp.float32)
            acc1 = jnp.zeros((_L,), jnp.float32)
            for k in range(_G):
                sv = sbuf[p, row, pl.ds(k * _L, _L)]
                rv = rbuf[p, row, pl.ds(k * _L, _L)]
                ov = obuf[p, row, pl.ds(k * _L, _L)]
                d = (sv - ov) + rv
                if k & 1:
                    acc1 = acc1 + d * d
                else:
                    acc0 = acc0 + d * d
            tot = _lane_total(acc0 + acc1)
            idxv = jnp.broadcast_to(c * _CH + row, (_L,)).astype(jnp.int32)
            plsc.store_scatter(res_v, [idxv], tot, mask=lanes == 0)

        if c == _NCHUNK // 2 - 1:
            wb.append(half_done(0))
        elif c == _NCHUNK - 1:
            wb.append(half_done(1))
    for h in wb:
        h.wait()


def kernel(pos_s, pos_r, pos_o, neg_s, neg_r, neg_o, e_embed, r_embed):
    i32 = jnp.int32
    out = _ttranse_sc(pos_s.astype(i32), pos_r.astype(i32), pos_o.astype(i32),
                      neg_s.astype(i32), neg_r.astype(i32), neg_o.astype(i32),
                      e_embed, r_embed)
    return out[0], out[1]


# speedup vs baseline: 1.0742x; 1.0742x over previous
import functools

import jax
import jax.numpy as jnp
from jax import lax
from jax.experimental import pallas as pl
from jax.experimental.pallas import tpu as pltpu
from jax.experimental.pallas import tpu_sc as plsc

_B = 16384
_DIM = 128
_NC = 2
_NS = 16
_NW = _NC * _NS
_HPW = _B // _NW
_RPW = 2 * _HPW
_CH = 128
_NCHUNK = _RPW // _CH
_L = 16
_G = _DIM // _L


def _lane_total(x):
    dnums = lax.GatherDimensionNumbers(
        offset_dims=(), collapsed_slice_dims=(0,), start_index_map=(0,))
    for sh in (8, 4, 2, 1):
        perm = lax.iota(jnp.int32, _L) ^ sh
        x = x + lax.gather(x, perm[:, None], dnums, slice_sizes=(1,),
                           mode=lax.GatherScatterMode.PROMISE_IN_BOUNDS)
    return x


def _sqrt16(x):
    i = plsc.bitcast(x, jnp.int32)
    g = plsc.bitcast(lax.shift_right_logical(i, 1) + 0x1FBD1DF5, jnp.float32)
    g = 0.5 * (g + x / g)
    g = 0.5 * (g + x / g)
    return g


@functools.partial(
    pl.kernel,
    mesh=plsc.VectorSubcoreMesh(core_axis_name="c", subcore_axis_name="s"),
    out_type=[jax.ShapeDtypeStruct((_B,), jnp.float32),
              jax.ShapeDtypeStruct((_B,), jnp.float32)],
    compiler_params=pltpu.CompilerParams(needs_layout_passes=False),
    scratch_types=[
        pltpu.VMEM((_RPW,), jnp.int32),
        pltpu.VMEM((_RPW,), jnp.int32),
        pltpu.VMEM((_RPW,), jnp.int32),
        pltpu.VMEM((2, _CH, _DIM), jnp.float32),
        pltpu.VMEM((2, _CH, _DIM), jnp.float32),
        pltpu.VMEM((2, _CH, _DIM), jnp.float32),
        pltpu.VMEM((_RPW,), jnp.float32),
        pltpu.SemaphoreType.DMA,
        pltpu.SemaphoreType.DMA,
        pltpu.SemaphoreType.DMA,
    ],
)
def _ttranse_sc(ps_hbm, pr_hbm, po_hbm, ns_hbm, nr_hbm, no_hbm,
                e_hbm, rel_hbm, pos_hbm, neg_hbm,
                sidx, ridx, oidx, sbuf, rbuf, obuf, res_v,
                sem0, sem1, semw):
    wid = lax.axis_index("s") * _NC + lax.axis_index("c")
    base = wid * _HPW
    sems = (sem0, sem1)
    ihp = [
        pltpu.async_copy(ps_hbm.at[pl.ds(base, _HPW)], sidx.at[pl.ds(0, _HPW)], sem0),
        pltpu.async_copy(pr_hbm.at[pl.ds(base, _HPW)], ridx.at[pl.ds(0, _HPW)], sem0),
        pltpu.async_copy(po_hbm.at[pl.ds(base, _HPW)], oidx.at[pl.ds(0, _HPW)], sem0),
    ]
    ihn = [
        pltpu.async_copy(ns_hbm.at[pl.ds(base, _HPW)], sidx.at[pl.ds(_HPW, _HPW)], semw),
        pltpu.async_copy(nr_hbm.at[pl.ds(base, _HPW)], ridx.at[pl.ds(_HPW, _HPW)], semw),
        pltpu.async_copy(no_hbm.at[pl.ds(base, _HPW)], oidx.at[pl.ds(_HPW, _HPW)], semw),
    ]
    for h in ihp:
        h.wait()
    lanes = lax.iota(jnp.int32, _L)

    def start(c):
        p = c & 1
        sem = sems[p]
        return (
            pltpu.async_copy(e_hbm.at[sidx.at[pl.ds(c * _CH, _CH)]],
                             sbuf.at[p], sem),
            pltpu.async_copy(rel_hbm.at[ridx.at[pl.ds(c * _CH, _CH)]],
                             rbuf.at[p], sem),
            pltpu.async_copy(e_hbm.at[oidx.at[pl.ds(c * _CH, _CH)]],
                             obuf.at[p], sem),
        )

    def half_done(h):
        @plsc.parallel_loop(0, _HPW // _L, unroll=4)
        def sqrt_body(i, h=h):
            off = h * _HPW + i * _L
            res_v[pl.ds(off, _L)] = _sqrt16(res_v[pl.ds(off, _L)])
        dst = pos_hbm if h == 0 else neg_hbm
        return pltpu.async_copy(res_v.at[pl.ds(h * _HPW, _HPW)],
                                dst.at[pl.ds(base, _HPW)], semw)

    wb = []
    pending = {0: start(0)}
    for c in range(_NCHUNK):
        p = c & 1
        if c + 1 < _NCHUNK:
            if c + 1 == _NCHUNK // 2:
                for h in ihn:
                    h.wait()
            pending[c + 1] = start(c + 1)
        for h in pending.pop(c):
            h.wait()

        @plsc.parallel_loop(0, _CH, unroll=3)
        def row_body(row, c=c, p=p):
            acc0 = jnp.zeros((_L,), jnp.float32)
            acc1 = jnp.zeros((_L,), jnp.float32)
            for k in range(_G):
                sv = sbuf[p, row, pl.ds(k * _L, _L)]
                rv = rbuf[p, row, pl.ds(k * _L, _L)]
                ov = obuf[p, row, pl.ds(k * _L, _L)]
                d = (sv - ov) + rv
                if k & 1:
                    acc1 = acc1 + d * d
                else:
                    acc0 = acc0 + d * d
            tot = _lane_total(acc0 + acc1)
            idxv = jnp.broadcast_to(c * _CH + row, (_L,)).astype(jnp.int32)
            plsc.store_scatter(res_v, [idxv], tot, mask=lanes == 0)

        if c == _NCHUNK // 2 - 1:
            wb.append(half_done(0))
        elif c == _NCHUNK - 1:
            wb.append(half_done(1))
    for h in wb:
        h.wait()


def kernel(pos_s, pos_r, pos_o, neg_s, neg_r, neg_o, e_embed, r_embed):
    i32 = jnp.int32
    out = _ttranse_sc(pos_s.astype(i32), pos_r.astype(i32), pos_o.astype(i32),
                      neg_s.astype(i32), neg_r.astype(i32), neg_o.astype(i32),
                      e_embed, r_embed)
    return out[0], out[1]
